# baseline (speedup 1.0000x reference)
import math

import jax
import jax.numpy as jnp
from jax import lax
from jax.experimental import pallas as pl
from jax.experimental.pallas import tpu as pltpu

D = 1024
H = 16
HD = 64
BS = 256
NB = 16
BATCH = 2
SINK_ITERS = 5
SCALE = HD ** -0.5
KA = D + 128


def _pack_w(w, b):
    wt = jnp.swapaxes(w, 0, 1).astype(jnp.bfloat16)
    brow = jnp.concatenate(
        [b.astype(jnp.bfloat16), jnp.zeros((KA - D - 1, D), jnp.bfloat16)], axis=0)
    return jnp.concatenate([wt, brow], axis=0)


def _perm_kernel(x_ref, wq_ref, bq_ref, wk_ref, bk_ref, wv_ref, bv_ref,
                 wo_ref, bo_ref, perm_ref, xbf0_ref, xbf1_ref, wbf_ref, xsum_ref):
    i = pl.program_id(0)
    xs0 = x_ref[:, 0, :]
    xs1 = x_ref[:, 1, :]
    xsum_ref[pl.ds(i, 1), :] = jnp.concatenate(
        [jnp.mean(xs0, axis=0, keepdims=True), jnp.mean(xs1, axis=0, keepdims=True)],
        axis=1)
    xbf0_ref[...] = xs0.astype(jnp.bfloat16)
    xbf1_ref[...] = xs1.astype(jnp.bfloat16)

    @pl.when(i == 0)
    def _():
        wbf_ref[0 * KA:1 * KA, :] = _pack_w(wq_ref[...] * SCALE, bq_ref[...] * SCALE)
        wbf_ref[1 * KA:2 * KA, :] = _pack_w(wk_ref[...], bk_ref[...])
        wbf_ref[2 * KA:3 * KA, :] = _pack_w(wv_ref[...], bv_ref[...])
        wbf_ref[3 * KA:4 * KA, :] = _pack_w(wo_ref[...], bo_ref[...])

    @pl.when(i == NB - 1)
    def _():
        inv_sqrt_d = 1.0 / math.sqrt(D)
        cols = []
        for bb in range(BATCH):
            xm = xsum_ref[:, bb * D:(bb + 1) * D]
            qb = lax.dot_general(xm, wq_ref[...], (((1,), (1,)), ((), ())),
                                 preferred_element_type=jnp.float32) + bq_ref[...]
            kb = lax.dot_general(xm, wk_ref[...], (((1,), (1,)), ((), ())),
                                 preferred_element_type=jnp.float32) + bk_ref[...]
            la = lax.dot_general(qb, kb, (((1,), (1,)), ((), ())),
                                 preferred_element_type=jnp.float32) * inv_sqrt_d
            for _ in range(SINK_ITERS):
                m1 = jnp.max(la, axis=1, keepdims=True)
                la = la - (m1 + jnp.log(jnp.sum(jnp.exp(la - m1), axis=1, keepdims=True)))
                m0 = jnp.max(la, axis=0, keepdims=True)
                la = la - (m0 + jnp.log(jnp.sum(jnp.exp(la - m0), axis=0, keepdims=True)))
            p = jnp.exp(la)
            mx = jnp.max(p, axis=1, keepdims=True)
            iota = lax.broadcasted_iota(jnp.int32, (NB, NB), 1)
            idx = jnp.min(jnp.where(p >= mx, iota, NB), axis=1, keepdims=True)
            cols.append(idx)
        perm_ref[...] = jnp.concatenate(cols, axis=1)


def _aug(nrows):
    return jnp.where(lax.broadcasted_iota(jnp.int32, (nrows, KA - D), 1) == 0,
                     1.0, 0.0).astype(jnp.bfloat16)


def _attention_block(xb, wbf_ref):
    xa = jnp.concatenate([xb, _aug(BS)], axis=1)
    q = lax.dot_general(xa, wbf_ref[0 * KA:1 * KA, :], (((1,), (0,)), ((), ())),
                        preferred_element_type=jnp.float32).astype(jnp.bfloat16)
    k = lax.dot_general(xa, wbf_ref[1 * KA:2 * KA, :], (((1,), (0,)), ((), ())),
                        preferred_element_type=jnp.float32).astype(jnp.bfloat16)
    v = lax.dot_general(xa, wbf_ref[2 * KA:3 * KA, :], (((1,), (0,)), ((), ())),
                        preferred_element_type=jnp.float32).astype(jnp.bfloat16)
    outs = []
    for h in range(H):
        qh = q[:, h * HD:(h + 1) * HD]
        kh = k[:, h * HD:(h + 1) * HD]
        vh = v[:, h * HD:(h + 1) * HD]
        st = lax.dot_general(kh, qh, (((1,), (1,)), ((), ())),
                             preferred_element_type=jnp.float32)
        m = jnp.max(st, axis=0, keepdims=True)
        e = jnp.exp(st - m)
        rsum = 1.0 / jnp.sum(e, axis=0, keepdims=True)
        p = (e * rsum).astype(jnp.bfloat16)
        acc = lax.dot_general(p, vh, (((0,), (0,)), ((), ())),
                              preferred_element_type=jnp.float32)
        outs.append(acc.astype(jnp.bfloat16))
    return jnp.concatenate(outs, axis=1)


def _fused_kernel(p_ref, xa_ref, xc_ref, wbf_ref, out_ref):
    del p_ref
    cat_a = _attention_block(xa_ref[...], wbf_ref)
    cat_c = _attention_block(xc_ref[...], wbf_ref)
    cat = jnp.concatenate([cat_a, cat_c], axis=0)
    cat = jnp.concatenate([cat, _aug(2 * BS)], axis=1)
    o = lax.dot_general(cat, wbf_ref[3 * KA:4 * KA, :], (((1,), (0,)), ((), ())),
                        preferred_element_type=jnp.float32)
    out_ref[...] = jnp.stack([o[:BS], o[BS:]], axis=1)


def kernel(x, Wq, bq, Wk, bk, Wv, bv, Wo, bo):
    S, B, Dd = x.shape
    assert (B, Dd) == (BATCH, D) and S == NB * BS

    bq2 = bq.reshape(1, D)
    bk2 = bk.reshape(1, D)
    bv2 = bv.reshape(1, D)
    bo2 = bo.reshape(1, D)

    perm2, xbf0, xbf1, wbf = pl.pallas_call(
        _perm_kernel,
        grid=(NB,),
        in_specs=[
            pl.BlockSpec((BS, B, D), lambda i: (i, 0, 0)),
            pl.BlockSpec((D, D), lambda i: (0, 0)),
            pl.BlockSpec((1, D), lambda i: (0, 0)),
            pl.BlockSpec((D, D), lambda i: (0, 0)),
            pl.BlockSpec((1, D), lambda i: (0, 0)),
            pl.BlockSpec((D, D), lambda i: (0, 0)),
            pl.BlockSpec((1, D), lambda i: (0, 0)),
            pl.BlockSpec((D, D), lambda i: (0, 0)),
            pl.BlockSpec((1, D), lambda i: (0, 0)),
        ],
        out_specs=[
            pl.BlockSpec((NB, B), lambda i: (0, 0)),
            pl.BlockSpec((BS, D), lambda i: (i, 0)),
            pl.BlockSpec((BS, D), lambda i: (i, 0)),
            pl.BlockSpec((4 * KA, D), lambda i: (0, 0)),
        ],
        out_shape=[
            jax.ShapeDtypeStruct((NB, B), jnp.int32),
            jax.ShapeDtypeStruct((S, D), jnp.bfloat16),
            jax.ShapeDtypeStruct((S, D), jnp.bfloat16),
            jax.ShapeDtypeStruct((4 * KA, D), jnp.bfloat16),
        ],
        scratch_shapes=[pltpu.VMEM((NB, B * D), jnp.float32)],
    )(x, Wq, bq2, Wk, bk2, Wv, bv2, Wo, bo2)

    grid_spec = pltpu.PrefetchScalarGridSpec(
        num_scalar_prefetch=1,
        grid=(NB,),
        in_specs=[
            pl.BlockSpec((BS, D), lambda t, p: (p[t, 0], 0)),
            pl.BlockSpec((BS, D), lambda t, p: (p[t, 1], 0)),
            pl.BlockSpec((4 * KA, D), lambda t, p: (0, 0)),
        ],
        out_specs=pl.BlockSpec((BS, B, D), lambda t, p: (t, 0, 0)),
    )
    out = pl.pallas_call(
        _fused_kernel,
        grid_spec=grid_spec,
        out_shape=jax.ShapeDtypeStruct((S, B, D), jnp.float32),
    )(perm2, xbf0, xbf1, wbf)

    return out


# speedup vs baseline: 1.2208x; 1.2208x over previous
import math

import jax
import jax.numpy as jnp
from jax import lax
from jax.experimental import pallas as pl
from jax.experimental.pallas import tpu as pltpu

D = 1024
H = 16
HD = 64
BS = 256
NB = 16
BATCH = 2
SINK_ITERS = 5
SCALE = HD ** -0.5


def _perm_kernel(x_ref, wq_ref, bq_ref, wk_ref, bk_ref, wv_ref, wo_ref,
                 perm_ref, xbf_ref, wbf_ref, xsum_ref):
    i = pl.program_id(0)
    xflat = jnp.concatenate([x_ref[:, 0, :], x_ref[:, 1, :]], axis=1)
    xsum_ref[pl.ds(i, 1), :] = jnp.mean(xflat, axis=0, keepdims=True)
    xbf_ref[...] = xflat.astype(jnp.bfloat16)

    @pl.when(i == 0)
    def _():
        wbf_ref[0 * D:1 * D, :] = (jnp.swapaxes(wq_ref[...], 0, 1) * SCALE).astype(jnp.bfloat16)
        wbf_ref[1 * D:2 * D, :] = jnp.swapaxes(wk_ref[...], 0, 1).astype(jnp.bfloat16)
        wbf_ref[2 * D:3 * D, :] = jnp.swapaxes(wv_ref[...], 0, 1).astype(jnp.bfloat16)
        wbf_ref[3 * D:4 * D, :] = jnp.swapaxes(wo_ref[...], 0, 1).astype(jnp.bfloat16)

    @pl.when(i == NB - 1)
    def _():
        inv_sqrt_d = 1.0 / math.sqrt(D)
        cols = []
        for bb in range(BATCH):
            xm = xsum_ref[:, bb * D:(bb + 1) * D]
            qb = lax.dot_general(xm, wq_ref[...], (((1,), (1,)), ((), ())),
                                 preferred_element_type=jnp.float32) + bq_ref[...]
            kb = lax.dot_general(xm, wk_ref[...], (((1,), (1,)), ((), ())),
                                 preferred_element_type=jnp.float32) + bk_ref[...]
            la = lax.dot_general(qb, kb, (((1,), (1,)), ((), ())),
                                 preferred_element_type=jnp.float32) * inv_sqrt_d
            for _ in range(SINK_ITERS):
                m1 = jnp.max(la, axis=1, keepdims=True)
                la = la - (m1 + jnp.log(jnp.sum(jnp.exp(la - m1), axis=1, keepdims=True)))
                m0 = jnp.max(la, axis=0, keepdims=True)
                la = la - (m0 + jnp.log(jnp.sum(jnp.exp(la - m0), axis=0, keepdims=True)))
            p = jnp.exp(la)
            mx = jnp.max(p, axis=1, keepdims=True)
            iota = lax.broadcasted_iota(jnp.int32, (NB, NB), 1)
            idx = jnp.min(jnp.where(p >= mx, iota, NB), axis=1, keepdims=True)
            cols.append(idx)
        perm_ref[...] = jnp.concatenate(cols, axis=1)


def _attention_block(xb, wbf_ref, bq, bk, bv):
    q = lax.dot_general(xb, wbf_ref[0 * D:1 * D, :], (((1,), (0,)), ((), ())),
                        preferred_element_type=jnp.float32)
    k = lax.dot_general(xb, wbf_ref[1 * D:2 * D, :], (((1,), (0,)), ((), ())),
                        preferred_element_type=jnp.float32)
    v = lax.dot_general(xb, wbf_ref[2 * D:3 * D, :], (((1,), (0,)), ((), ())),
                        preferred_element_type=jnp.float32)
    q = (q + bq).astype(jnp.bfloat16)
    k = (k + bk).astype(jnp.bfloat16)
    v = (v + bv).astype(jnp.bfloat16)
    outs = []
    for h in range(H):
        qh = q[:, h * HD:(h + 1) * HD]
        kh = k[:, h * HD:(h + 1) * HD]
        vh = v[:, h * HD:(h + 1) * HD]
        st = lax.dot_general(kh, qh, (((1,), (1,)), ((), ())),
                             preferred_element_type=jnp.float32)
        m = jnp.max(st, axis=0, keepdims=True)
        e = jnp.exp(st - m)
        rsum = 1.0 / jnp.sum(e, axis=0, keepdims=True)
        p = (e * rsum).astype(jnp.bfloat16)
        acc = lax.dot_general(p, vh, (((0,), (0,)), ((), ())),
                              preferred_element_type=jnp.float32)
        outs.append(acc.astype(jnp.bfloat16))
    return jnp.concatenate(outs, axis=1)


def _fused_kernel(p_ref, xa_ref, xc_ref, wbf_ref,
                  bq_ref, bk_ref, bv_ref, bo_ref, out_ref):
    del p_ref
    bq = bq_ref[...] * SCALE
    bk = bk_ref[...]
    bv = bv_ref[...]
    cat_a = _attention_block(xa_ref[...], wbf_ref, bq, bk, bv)
    cat_c = _attention_block(xc_ref[...], wbf_ref, bq, bk, bv)
    cat = jnp.concatenate([cat_a, cat_c], axis=0)
    wo = wbf_ref[3 * D:4 * D, :]
    o = lax.dot_general(cat, wo, (((1,), (0,)), ((), ())),
                        preferred_element_type=jnp.float32) + bo_ref[...]
    out_ref[...] = jnp.stack([o[:BS], o[BS:]], axis=1)


def kernel(x, Wq, bq, Wk, bk, Wv, bv, Wo, bo):
    S, B, Dd = x.shape
    assert (B, Dd) == (BATCH, D) and S == NB * BS

    bq2 = bq.reshape(1, D)
    bk2 = bk.reshape(1, D)
    bv2 = bv.reshape(1, D)
    bo2 = bo.reshape(1, D)

    perm2, xbf, wbf = pl.pallas_call(
        _perm_kernel,
        grid=(NB,),
        in_specs=[
            pl.BlockSpec((BS, B, D), lambda i: (i, 0, 0)),
            pl.BlockSpec((D, D), lambda i: (0, 0)),
            pl.BlockSpec((1, D), lambda i: (0, 0)),
            pl.BlockSpec((D, D), lambda i: (0, 0)),
            pl.BlockSpec((1, D), lambda i: (0, 0)),
            pl.BlockSpec((D, D), lambda i: (0, 0)),
            pl.BlockSpec((D, D), lambda i: (0, 0)),
        ],
        out_specs=[
            pl.BlockSpec((NB, B), lambda i: (0, 0)),
            pl.BlockSpec((BS, B * D), lambda i: (i, 0)),
            pl.BlockSpec((4 * D, D), lambda i: (0, 0)),
        ],
        out_shape=[
            jax.ShapeDtypeStruct((NB, B), jnp.int32),
            jax.ShapeDtypeStruct((S, B * D), jnp.bfloat16),
            jax.ShapeDtypeStruct((4 * D, D), jnp.bfloat16),
        ],
        scratch_shapes=[pltpu.VMEM((NB, B * D), jnp.float32)],
    )(x, Wq, bq2, Wk, bk2, Wv, Wo)

    grid_spec = pltpu.PrefetchScalarGridSpec(
        num_scalar_prefetch=1,
        grid=(NB,),
        in_specs=[
            pl.BlockSpec((BS, D), lambda t, p: (p[t, 0], 0)),
            pl.BlockSpec((BS, D), lambda t, p: (p[t, 1], 1)),
            pl.BlockSpec((4 * D, D), lambda t, p: (0, 0)),
            pl.BlockSpec((1, D), lambda t, p: (0, 0)),
            pl.BlockSpec((1, D), lambda t, p: (0, 0)),
            pl.BlockSpec((1, D), lambda t, p: (0, 0)),
            pl.BlockSpec((1, D), lambda t, p: (0, 0)),
        ],
        out_specs=pl.BlockSpec((BS, B, D), lambda t, p: (t, 0, 0)),
    )
    out = pl.pallas_call(
        _fused_kernel,
        grid_spec=grid_spec,
        out_shape=jax.ShapeDtypeStruct((S, B, D), jnp.float32),
    )(perm2, xbf, xbf, wbf, bq2, bk2, bv2, bo2)

    return out
